# baseline (speedup 1.0000x reference)
import jax
import jax.numpy as jnp
from jax import lax
from jax.experimental import pallas as pl
from jax.experimental.pallas import tpu as pltpu
from jax.experimental.pallas import tpu_sc as plsc

_B = 16384
_D = 32
_NC = 2
_NS = 16
_NW = _NC * _NS
_BPW = _B // _NW
_CHUNK = 128
_NCHUNK = _BPW // _CHUNK
_GROUPS = _BPW // 16


def _sc_body(user_ref, item_ref, ut_ref, it_ref, par_ref, out_ref,
             idx_u, idx_i, rows_u, rows_i, out_v, w_v, sem):
    wid = lax.axis_index("s") * _NC + lax.axis_index("c")
    pltpu.sync_copy(user_ref.at[wid], idx_u)
    pltpu.sync_copy(item_ref.at[wid], idx_i)
    pltpu.sync_copy(par_ref, w_v)
    copies = []
    for j in range(_NCHUNK):
        sl = pl.ds(j * _CHUNK, _CHUNK)
        copies.append(pltpu.async_copy(ut_ref.at[idx_u.at[j]], rows_u.at[sl], sem))
        copies.append(pltpu.async_copy(it_ref.at[idx_i.at[j]], rows_i.at[sl], sem))
    for c in copies:
        c.wait()

    wu0 = w_v[pl.ds(0, 16)]
    wu1 = w_v[pl.ds(16, 16)]
    wi0 = w_v[pl.ds(32, 16)]
    wi1 = w_v[pl.ds(48, 16)]
    bias = w_v[pl.ds(64, 16)]
    lane = jnp.arange(16, dtype=jnp.int32)

    def group(g, carry):
        row0 = g * 16
        acc = bias
        for r in range(16):
            b = row0 + r
            t = (rows_u[b, pl.ds(0, 16)] * wu0 +
                 rows_u[b, pl.ds(16, 16)] * wu1 +
                 rows_i[b, pl.ds(0, 16)] * wi0 +
                 rows_i[b, pl.ds(16, 16)] * wi1)
            acc = jnp.where(lane == r, acc + jnp.sum(t), acc)
        out_v[pl.ds(row0, 16)] = 1.0 / (1.0 + jnp.exp(-acc))
        return carry

    lax.fori_loop(0, _GROUPS, group, 0)
    pltpu.sync_copy(out_v, out_ref.at[pl.ds(wid * _BPW, _BPW)])


_sc_call = pl.kernel(
    _sc_body,
    out_type=jax.ShapeDtypeStruct((_B,), jnp.float32),
    mesh=plsc.VectorSubcoreMesh(core_axis_name="c", subcore_axis_name="s"),
    scratch_types=[
        pltpu.VMEM((_NCHUNK, _CHUNK), jnp.int32),
        pltpu.VMEM((_NCHUNK, _CHUNK), jnp.int32),
        pltpu.VMEM((_BPW, _D), jnp.float32),
        pltpu.VMEM((_BPW, _D), jnp.float32),
        pltpu.VMEM((_BPW,), jnp.float32),
        pltpu.VMEM((80,), jnp.float32),
        pltpu.SemaphoreType.DMA,
    ],
    compiler_params=pltpu.CompilerParams(
        needs_layout_passes=False, use_tc_tiling_on_sc=False),
)


def kernel(user, item, user_table, item_table, fc_w, fc_b):
    u3 = user.reshape(_NW, _NCHUNK, _CHUNK)
    i3 = item.reshape(_NW, _NCHUNK, _CHUNK)
    params = jnp.concatenate(
        [fc_w.reshape(-1), jnp.broadcast_to(fc_b.reshape(1), (16,))])
    out = _sc_call(u3, i3, user_table, item_table, params)
    return out.reshape(_B, 1)
